# baseline (speedup 1.0000x reference)
import functools

import jax
import jax.numpy as jnp
from jax import lax
from jax.experimental import pallas as pl
from jax.experimental.pallas import tpu as pltpu
from jax.experimental.pallas import tpu_sc as plsc

S = 13
NCELL = S * S
NCH = 25
BATCH = 256
NOBJ = 8
NC, NS, L = 2, 16, 16
NW = NC * NS
NSLOT = 6
CLAMP = float(S - 1e-6)

_mesh = plsc.VectorSubcoreMesh(core_axis_name="c", subcore_axis_name="s")

_cell_scratch = []
for _ in range(NSLOT):
    _cell_scratch.append(pltpu.VMEM((NCH, 2, 128), jnp.float32))
    _cell_scratch.append(pltpu.VMEM((2, 128), jnp.float32))


@functools.partial(
    pl.kernel,
    out_type=jax.ShapeDtypeStruct((NW, L), jnp.float32),
    mesh=_mesh,
    scratch_types=[
        *_cell_scratch,
        pltpu.VMEM((NOBJ, 2, 4, 128), jnp.float32),
        pltpu.VMEM((NOBJ, 2, 128), jnp.int32),
        pltpu.VMEM((NOBJ, 2, 128), jnp.float32),
        pltpu.VMEM((NOBJ, 2, 128), jnp.float32),
        pltpu.VMEM((NOBJ, 2, 128), jnp.float32),
        pltpu.VMEM((NOBJ, 2, 128), jnp.float32),
        pltpu.VMEM((L,), jnp.float32),
        pltpu.SemaphoreType.DMA,
    ],
)
def _yolo_sc(pred_hbm, box_hbm, out_hbm, *refs):
    cbs = refs[0:2 * NSLOT:2]
    cfs = refs[1:2 * NSLOT:2]
    (box_b, cid_s, tx_s, ty_s, tw_s, th_s, acc_r, sem) = refs[2 * NSLOT:]
    wid = lax.axis_index("s") * NC + lax.axis_index("c")

    iota = lax.iota(jnp.int32, L)
    zero = jnp.float32(0.0) * iota.astype(jnp.float32)

    for slot in range(NSLOT):
        cid = slot * NW + wid

        @pl.when(cid < NCELL)
        def _():
            pltpu.async_copy(pred_hbm.at[cid // S, cid % S, :, :, 0],
                             cbs[slot], sem)
            pltpu.async_copy(pred_hbm.at[cid // S, cid % S, 4, :, 1],
                             cfs[slot], sem)

        if slot == NSLOT - 1:
            @pl.when(cid >= NCELL)
            def _zero_fill():
                for ch in range(NCH):
                    for bh in range(2):
                        for k in range(8):
                            cbs[slot][ch, bh, pl.ds(k * L, L)] = zero
                for bh in range(2):
                    for k in range(8):
                        cfs[slot][bh, pl.ds(k * L, L)] = zero

    pltpu.sync_copy(box_hbm, box_b)
    acc_r[...] = zero

    def geom_body(i, _):
        j = i >> 4
        bh = (i >> 3) & 1
        bl0 = (i & 7) * L
        x1 = box_b[j, bh, 0, pl.ds(bl0, L)]
        y1 = box_b[j, bh, 1, pl.ds(bl0, L)]
        x2 = box_b[j, bh, 2, pl.ds(bl0, L)]
        y2 = box_b[j, bh, 3, pl.ds(bl0, L)]
        x = jnp.minimum(((x1 + x2) / 2.0) / 32.0, CLAMP)
        y = jnp.minimum(((y1 + y2) / 2.0) / 32.0, CLAMP)
        gxi = x.astype(jnp.int32)
        gyi = y.astype(jnp.int32)
        cid_s[j, bh, pl.ds(bl0, L)] = gyi * S + gxi
        tx_s[j, bh, pl.ds(bl0, L)] = x - gxi.astype(jnp.float32)
        ty_s[j, bh, pl.ds(bl0, L)] = y - gyi.astype(jnp.float32)
        tw_s[j, bh, pl.ds(bl0, L)] = (x2 - x1) / 416.0
        th_s[j, bh, pl.ds(bl0, L)] = (y2 - y1) / 416.0
        return 0

    lax.fori_loop(0, NOBJ * 2 * NOBJ, geom_body, 0)

    for slot in range(NSLOT):
        cid = slot * NW + wid

        @pl.when(cid < NCELL)
        def _drain():
            pltpu.make_async_copy(pred_hbm.at[cid // S, cid % S, :, :, 0],
                                  cbs[slot], sem).wait()
            pltpu.make_async_copy(pred_hbm.at[cid // S, cid % S, 4, :, 1],
                                  cfs[slot], sem).wait()

    def bg_body(bg, acc):
        bh = bg >> 3
        bl0 = (bg & 7) * L
        cj = [cid_s[j, bh, pl.ds(bl0, L)] for j in range(NOBJ)]
        txj = [tx_s[j, bh, pl.ds(bl0, L)] for j in range(NOBJ)]
        tyj = [ty_s[j, bh, pl.ds(bl0, L)] for j in range(NOBJ)]
        twj = [tw_s[j, bh, pl.ds(bl0, L)] for j in range(NOBJ)]
        thj = [th_s[j, bh, pl.ds(bl0, L)] for j in range(NOBJ)]
        for slot in range(NSLOT):
            cid = slot * NW + wid
            cell_b = cbs[slot]
            conf_b = cfs[slot]
            v0 = cell_b[4, bh, pl.ds(bl0, L)]
            v1 = conf_b[bh, pl.ds(bl0, L)]
            acc = acc + v0 * v0 + v1 * v1
            mask = iota < 0
            tx = zero
            ty = zero
            tw = zero
            th = zero
            for j in range(NOBJ):
                m = cj[j] == cid
                mask = mask | m
                tx = jnp.where(m, txj[j], tx)
                ty = jnp.where(m, tyj[j], ty)
                tw = jnp.where(m, twj[j], tw)
                th = jnp.where(m, thj[j], th)
            coord = zero
            cls = zero
            for ch in range(NCH):
                v = cell_b[ch, bh, pl.ds(bl0, L)]
                if ch == 0:
                    d = v - tx
                    coord = coord + d * d
                elif ch == 1:
                    d = v - ty
                    coord = coord + d * d
                elif ch == 2:
                    d = v - tw
                    coord = coord + d * d
                elif ch == 3:
                    d = v - th
                    coord = coord + d * d
                elif ch == 4:
                    conf_c = 1.0 - 2.0 * v
                else:
                    cls = cls + v * v
            corr = 5.0 * coord + conf_c + cls
            acc = acc + jnp.where(mask, corr, 0.0)
        return acc

    total = lax.fori_loop(0, 16, bg_body, zero)
    acc_r[...] = acc_r[...] + total
    pltpu.sync_copy(acc_r, out_hbm.at[wid])


def kernel(predictions, boxes, labels):
    pred6 = (predictions.reshape(2, 128, S, S, 2, NCH)
                        .transpose(2, 3, 5, 0, 4, 1))
    box4 = (boxes.reshape(2, 128, NOBJ, 4)
                 .transpose(2, 0, 3, 1))
    partials = _yolo_sc(pred6, box4)
    return jnp.sum(partials) / predictions.shape[0]


# speedup vs baseline: 1.2185x; 1.2185x over previous
import functools

import jax
import jax.numpy as jnp
from jax import lax
from jax.experimental import pallas as pl
from jax.experimental.pallas import tpu as pltpu
from jax.experimental.pallas import tpu_sc as plsc

S = 13
NCELL = S * S
NCH = 25
BATCH = 256
NOBJ = 8
NC, NS, L = 2, 16, 16
NW = NC * NS
NSLOT = 6
CLAMP = float(S - 1e-6)

_mesh = plsc.VectorSubcoreMesh(core_axis_name="c", subcore_axis_name="s")


@functools.partial(
    pl.kernel,
    out_type=jax.ShapeDtypeStruct((NW, L), jnp.float32),
    mesh=_mesh,
    scratch_types=[
        pltpu.VMEM((NCH, 2, 2, 128), jnp.float32),
        pltpu.VMEM((NCH, 2, 2, 128), jnp.float32),
        pltpu.VMEM((NOBJ, 2, 4, 128), jnp.float32),
        pltpu.VMEM((NOBJ, 2, 128), jnp.int32),
        pltpu.VMEM((NOBJ, 2, 128), jnp.float32),
        pltpu.VMEM((NOBJ, 2, 128), jnp.float32),
        pltpu.VMEM((NOBJ, 2, 128), jnp.float32),
        pltpu.VMEM((NOBJ, 2, 128), jnp.float32),
        pltpu.VMEM((L,), jnp.float32),
        pltpu.SemaphoreType.DMA,
        pltpu.SemaphoreType.DMA,
    ],
)
def _yolo_sc(pred_hbm, box_hbm, out_hbm, cb0, cb1, box_b, cid_s, tx_s, ty_s,
             tw_s, th_s, acc_r, sem0, sem1):
    wid = lax.axis_index("s") * NC + lax.axis_index("c")
    bufs = (cb0, cb1)
    sems = (sem0, sem1)

    def start_fetch(slot):
        cid = slot * NW + wid

        @pl.when(cid < NCELL)
        def _():
            pltpu.async_copy(pred_hbm.at[cid // S, cid % S],
                             bufs[slot % 2], sems[slot % 2])

    start_fetch(0)
    pltpu.sync_copy(box_hbm, box_b)

    iota = lax.iota(jnp.int32, L)
    zero = jnp.float32(0.0) * iota.astype(jnp.float32)
    acc_r[...] = zero

    def geom_body(i, _):
        j = i >> 4
        bh = (i >> 3) & 1
        bl0 = (i & 7) * L
        x1 = box_b[j, bh, 0, pl.ds(bl0, L)]
        y1 = box_b[j, bh, 1, pl.ds(bl0, L)]
        x2 = box_b[j, bh, 2, pl.ds(bl0, L)]
        y2 = box_b[j, bh, 3, pl.ds(bl0, L)]
        x = jnp.minimum(((x1 + x2) / 2.0) / 32.0, CLAMP)
        y = jnp.minimum(((y1 + y2) / 2.0) / 32.0, CLAMP)
        gxi = x.astype(jnp.int32)
        gyi = y.astype(jnp.int32)
        cid_s[j, bh, pl.ds(bl0, L)] = gyi * S + gxi
        tx_s[j, bh, pl.ds(bl0, L)] = x - gxi.astype(jnp.float32)
        ty_s[j, bh, pl.ds(bl0, L)] = y - gyi.astype(jnp.float32)
        tw_s[j, bh, pl.ds(bl0, L)] = (x2 - x1) / 416.0
        th_s[j, bh, pl.ds(bl0, L)] = (y2 - y1) / 416.0
        return 0

    lax.fori_loop(0, NOBJ * 2 * NOBJ, geom_body, 0)

    for slot in range(NSLOT):
        cid = slot * NW + wid
        if slot + 1 < NSLOT:
            start_fetch(slot + 1)

        @pl.when(cid < NCELL)
        def _process():
            cell_b = bufs[slot % 2]
            pltpu.make_async_copy(pred_hbm.at[cid // S, cid % S],
                                  cell_b, sems[slot % 2]).wait()

            def bg_body(bg, acc):
                bh = bg >> 3
                bl0 = (bg & 7) * L
                v0 = cell_b[4, bh, 0, pl.ds(bl0, L)]
                v1 = cell_b[4, bh, 1, pl.ds(bl0, L)]
                acc = acc + v0 * v0 + v1 * v1
                mask = iota < 0
                tx = zero
                ty = zero
                tw = zero
                th = zero
                for j in range(NOBJ):
                    cj = cid_s[j, bh, pl.ds(bl0, L)]
                    m = cj == cid
                    mask = mask | m
                    tx = jnp.where(m, tx_s[j, bh, pl.ds(bl0, L)], tx)
                    ty = jnp.where(m, ty_s[j, bh, pl.ds(bl0, L)], ty)
                    tw = jnp.where(m, tw_s[j, bh, pl.ds(bl0, L)], tw)
                    th = jnp.where(m, th_s[j, bh, pl.ds(bl0, L)], th)
                coord = zero
                cls = zero
                for ch in range(NCH):
                    v = cell_b[ch, bh, 0, pl.ds(bl0, L)]
                    if ch == 0:
                        d = v - tx
                        coord = coord + d * d
                    elif ch == 1:
                        d = v - ty
                        coord = coord + d * d
                    elif ch == 2:
                        d = v - tw
                        coord = coord + d * d
                    elif ch == 3:
                        d = v - th
                        coord = coord + d * d
                    elif ch == 4:
                        conf_c = 1.0 - 2.0 * v
                    else:
                        cls = cls + v * v
                corr = 5.0 * coord + conf_c + cls
                return acc + jnp.where(mask, corr, 0.0)

            total = lax.fori_loop(0, 16, bg_body, zero)
            acc_r[...] = acc_r[...] + total

    pltpu.sync_copy(acc_r, out_hbm.at[wid])


def kernel(predictions, boxes, labels):
    pred6 = (predictions.reshape(2, 128, S, S, 2, NCH)
                        .transpose(2, 3, 5, 0, 4, 1))
    box4 = (boxes.reshape(2, 128, NOBJ, 4)
                 .transpose(2, 0, 3, 1))
    partials = _yolo_sc(pred6, box4)
    return jnp.sum(partials) / predictions.shape[0]
